# Initial kernel scaffold; baseline (speedup 1.0000x reference)
import jax
import jax.numpy as jnp
from jax.experimental import pallas as pl


def kernel(x, edge_index, target, gamma, beta, W1, a1s, a1d, b1, W2, a2s, a2d, b2, Wp, bp, Wd, bd):
    raise NotImplementedError("write your pallas kernel here")


# speedup vs baseline: 1.3828x; 1.3828x over previous
import functools

import jax
import jax.numpy as jnp
from jax.experimental import pallas as pl
from jax.experimental.pallas import tpu as pltpu

N = 10000
E = 64000
D = 768
IN = 3 * D
CLS = 10
BN_EPS = 1e-5
NEG = 0.2

RB = 1000
WCAT = 1920


def _mm_kernel(x_ref, w_ref, r_ref, o_ref):
    o_ref[...] = (jnp.dot(x_ref[...], w_ref[...],
                          preferred_element_type=jnp.float32) + r_ref[...])


def _mm_add_kernel(a_ref, w_ref, c_ref, o_ref):
    o_ref[...] = (jnp.dot(a_ref[...], w_ref[...],
                          preferred_element_type=jnp.float32) + c_ref[...])


def _matmul_row(x, w, row):
    n, k = x.shape
    m = w.shape[1]
    return pl.pallas_call(
        _mm_kernel,
        grid=(n // RB,),
        in_specs=[
            pl.BlockSpec((RB, k), lambda i: (i, 0)),
            pl.BlockSpec((k, m), lambda i: (0, 0)),
            pl.BlockSpec((1, m), lambda i: (0, 0)),
        ],
        out_specs=pl.BlockSpec((RB, m), lambda i: (i, 0)),
        out_shape=jax.ShapeDtypeStruct((n, m), jnp.float32),
    )(x, w, row)


def _matmul_addc(a, w, c):
    n, k = a.shape
    m = w.shape[1]
    return pl.pallas_call(
        _mm_add_kernel,
        grid=(n // RB,),
        in_specs=[
            pl.BlockSpec((RB, k), lambda i: (i, 0)),
            pl.BlockSpec((k, m), lambda i: (0, 0)),
            pl.BlockSpec((RB, m), lambda i: (i, 0)),
        ],
        out_specs=pl.BlockSpec((RB, m), lambda i: (i, 0)),
        out_shape=jax.ShapeDtypeStruct((n, m), jnp.float32),
    )(a, w, c)


def _edge_phase(h, a_src_n, a_dst_n, src, dst, b):
    n = h.shape[0]
    e = a_src_n[src] + a_dst_n[dst]
    e = jnp.where(e > 0, e, NEG * e)
    es = a_src_n + a_dst_n
    es = jnp.where(es > 0, es, NEG * es)
    emax = jnp.maximum(jax.ops.segment_max(e, dst, num_segments=n), es)
    ex = jnp.exp(e - emax[dst])
    exs = jnp.exp(es - emax)
    den = jax.ops.segment_sum(ex, dst, num_segments=n) + exs
    out = jax.ops.segment_sum((ex / den[dst])[:, None] * h[src], dst,
                              num_segments=n)
    return out + (exs / den)[:, None] * h + b


def _ce(logits, target):
    mask = (target >= 0)
    tc = jnp.where(mask, target, 0)
    lp = jax.nn.log_softmax(logits, axis=-1)
    nll = -jnp.take_along_axis(lp, tc[:, None], axis=1)[:, 0]
    m = mask.astype(jnp.float32)
    return jnp.sum(nll * m) / jnp.maximum(jnp.sum(m), 1.0)


def kernel(x, edge_index, target, gamma, beta, W1, a1s, a1d, b1,
           W2, a2s, a2d, b2, Wp, bp, Wd, bd):
    src, dst = edge_index[0], edge_index[1]

    mean = jnp.mean(x, axis=0)
    var = jnp.var(x, axis=0)
    s = gamma * jax.lax.rsqrt(var + BN_EPS)
    t = beta - mean * s

    W2a, W2b = W2[:D], W2[D:]
    wcat = jnp.zeros((IN, WCAT), jnp.float32)
    wcat = wcat.at[:, 0:D].set(W1)
    wcat = wcat.at[:, D].set(W1 @ a1s)
    wcat = wcat.at[:, D + 1].set(W1 @ a1d)
    wcat = wcat.at[:, 896:896 + D].set(W2b)
    wcat = wcat.at[:, 896 + D].set(W2b @ a2s)
    wcat = wcat.at[:, 896 + D + 1].set(W2b @ a2d)
    wcat = wcat.at[:, 1792:1792 + CLS].set(Wd)
    wcat = s[:, None] * wcat

    tW1 = t @ W1
    tW2b = t @ W2b
    row = jnp.zeros((1, WCAT), jnp.float32)
    row = row.at[0, 0:D].set(tW1)
    row = row.at[0, D].set(tW1 @ a1s)
    row = row.at[0, D + 1].set(tW1 @ a1d)
    row = row.at[0, 896:896 + D].set(tW2b)
    row = row.at[0, 896 + D].set(tW2b @ a2s)
    row = row.at[0, 896 + D + 1].set(tW2b @ a2d)
    row = row.at[0, 1792:1792 + CLS].set(t @ Wd + bd)

    P = _matmul_row(x, wcat, row)

    h1 = P[:, 0:D]
    out1 = _edge_phase(h1, P[:, D], P[:, D + 1], src, dst, b1)

    w3 = jnp.zeros((D, 896), jnp.float32)
    w3 = w3.at[:, 0:D].set(W2a)
    w3 = w3.at[:, D].set(W2a @ a2s)
    w3 = w3.at[:, D + 1].set(W2a @ a2d)
    O3 = _matmul_addc(out1, w3, P[:, 896:1792])

    h2 = O3[:, 0:D]
    out = _edge_phase(h2, O3[:, D], O3[:, D + 1], src, dst, b2)

    direct = P[:, 1792:1792 + CLS]

    wp = jnp.zeros((D, 128), jnp.float32)
    wp = wp.at[:, 0:CLS].set(Wp)
    rowp = jnp.zeros((1, 128), jnp.float32).at[0, 0:CLS].set(bp)
    pooler = _matmul_row(out, wp, rowp)[:, 0:CLS]

    loss = _ce(pooler, target) + _ce(direct, target)
    return (out, direct, pooler, loss)


# speedup vs baseline: 1.6220x; 1.1730x over previous
import functools

import jax
import jax.numpy as jnp
from jax import lax
from jax.experimental import pallas as pl
from jax.experimental.pallas import tpu as pltpu

N = 10000
E = 64000
D = 768
IN = 3 * D
CLS = 10
BN_EPS = 1e-5
NEG = 0.2

RB = 1000
WCAT = 1792


def _edge_phase(h, asn, adn, src, dst):
    e = asn[src] + adn[dst]
    e = jnp.where(e > 0, e, NEG * e)
    es = asn + adn
    es = jnp.where(es > 0, es, NEG * es)
    m = jnp.maximum(jnp.max(e), jnp.max(es))
    ex = jnp.exp(e - m)
    exs = jnp.exp(es - m)
    den = jax.ops.segment_sum(ex, dst, num_segments=N) + exs
    agg = jax.ops.segment_sum(ex[:, None] * h[src], dst,
                              num_segments=N) + exs[:, None] * h
    return agg, (1.0 / den)[:, None]


def _mm_split_kernel(x_ref, w_ref, r_ref, o1_ref, o2_ref, o3_ref):
    p = jnp.dot(x_ref[...], w_ref[...],
                preferred_element_type=jnp.float32) + r_ref[...]
    o1_ref[...] = p[:, :D]
    o2_ref[...] = p[:, D:D + 128]
    o3_ref[...] = p[:, D + 128:]


def _l2_kernel(p_ref, iv_ref, w_ref, c1_ref, c2_ref, oh_ref, oa_ref):
    a = p_ref[...] * iv_ref[...]
    p = jnp.dot(a, w_ref[...], preferred_element_type=jnp.float32)
    oh_ref[...] = p[:, :D] + c1_ref[...]
    oa_ref[...] = p[:, D:] + c2_ref[...]


def _final_kernel(q_ref, iv_ref, b2_ref, wp_ref, bp_ref, out_ref, pool_ref):
    o = q_ref[...] * iv_ref[...] + b2_ref[...]
    out_ref[...] = o
    pool_ref[...] = jnp.dot(o, wp_ref[...],
                            preferred_element_type=jnp.float32) + bp_ref[...]


def _ce(logits, target):
    mask = (target >= 0)
    tc = jnp.where(mask, target, 0)
    lp = jax.nn.log_softmax(logits, axis=-1)
    nll = -jnp.take_along_axis(lp, tc[:, None], axis=1)[:, 0]
    m = mask.astype(jnp.float32)
    return jnp.sum(nll * m) / jnp.maximum(jnp.sum(m), 1.0)


def kernel(x, edge_index, target, gamma, beta, W1, a1s, a1d, b1,
           W2, a2s, a2d, b2, Wp, bp, Wd, bd):
    src, dst = edge_index[0], edge_index[1]

    mean = jnp.mean(x, axis=0)
    var = jnp.var(x, axis=0)
    s = gamma * lax.rsqrt(var + BN_EPS)
    t = beta - mean * s

    W2a, W2b = W2[:D], W2[D:]
    v1s, v1d = W1 @ a1s, W1 @ a1d
    v2s, v2d = W2b @ a2s, W2b @ a2d
    w3s, w3d = W2a @ a2s, W2a @ a2d

    wcat = jnp.zeros((IN, WCAT), jnp.float32)
    wcat = wcat.at[:, 0:D].set(W1)
    wcat = wcat.at[:, D].set(v1s)
    wcat = wcat.at[:, D + 1].set(v1d)
    wcat = wcat.at[:, D + 2:D + 2 + CLS].set(Wd)
    wcat = wcat.at[:, 896:896 + D].set(W2b)
    wcat = wcat.at[:, 896 + D].set(v2s)
    wcat = wcat.at[:, 896 + D + 1].set(v2d)
    wcat = s[:, None] * wcat

    tW1 = t @ W1
    tW2b = t @ W2b
    row = jnp.zeros((1, WCAT), jnp.float32)
    row = row.at[0, 0:D].set(tW1)
    row = row.at[0, D].set(tW1 @ a1s)
    row = row.at[0, D + 1].set(tW1 @ a1d)
    row = row.at[0, D + 2:D + 2 + CLS].set(t @ Wd + bd)
    row = row.at[0, 896:896 + D].set(tW2b + b1 @ W2a)
    row = row.at[0, 896 + D].set(tW2b @ a2s + b1 @ w3s)
    row = row.at[0, 896 + D + 1].set(tW2b @ a2d + b1 @ w3d)

    h1, small, mid = pl.pallas_call(
        _mm_split_kernel,
        grid=(N // RB,),
        in_specs=[
            pl.BlockSpec((RB, IN), lambda i: (i, 0)),
            pl.BlockSpec((IN, WCAT), lambda i: (0, 0)),
            pl.BlockSpec((1, WCAT), lambda i: (0, 0)),
        ],
        out_specs=[
            pl.BlockSpec((RB, D), lambda i: (i, 0)),
            pl.BlockSpec((RB, 128), lambda i: (i, 0)),
            pl.BlockSpec((RB, 896), lambda i: (i, 0)),
        ],
        out_shape=[
            jax.ShapeDtypeStruct((N, D), jnp.float32),
            jax.ShapeDtypeStruct((N, 128), jnp.float32),
            jax.ShapeDtypeStruct((N, 896), jnp.float32),
        ],
    )(x, wcat, row)

    agg1, invd1 = _edge_phase(h1, small[:, 0], small[:, 1], src, dst)

    w3 = jnp.zeros((D, 896), jnp.float32)
    w3 = w3.at[:, 0:D].set(W2a)
    w3 = w3.at[:, D].set(w3s)
    w3 = w3.at[:, D + 1].set(w3d)

    h2, al2 = pl.pallas_call(
        _l2_kernel,
        grid=(N // RB,),
        in_specs=[
            pl.BlockSpec((RB, D), lambda i: (i, 0)),
            pl.BlockSpec((RB, 1), lambda i: (i, 0)),
            pl.BlockSpec((D, 896), lambda i: (0, 0)),
            pl.BlockSpec((RB, D), lambda i: (i, 0)),
            pl.BlockSpec((RB, 128), lambda i: (i, 0)),
        ],
        out_specs=[
            pl.BlockSpec((RB, D), lambda i: (i, 0)),
            pl.BlockSpec((RB, 128), lambda i: (i, 0)),
        ],
        out_shape=[
            jax.ShapeDtypeStruct((N, D), jnp.float32),
            jax.ShapeDtypeStruct((N, 128), jnp.float32),
        ],
    )(agg1, invd1, w3, mid[:, :D], mid[:, D:])

    agg2, invd2 = _edge_phase(h2, al2[:, 0], al2[:, 1], src, dst)

    wp = jnp.zeros((D, 128), jnp.float32).at[:, 0:CLS].set(Wp)
    bprow = jnp.zeros((1, 128), jnp.float32).at[0, 0:CLS].set(bp)

    out, pooler128 = pl.pallas_call(
        _final_kernel,
        grid=(N // RB,),
        in_specs=[
            pl.BlockSpec((RB, D), lambda i: (i, 0)),
            pl.BlockSpec((RB, 1), lambda i: (i, 0)),
            pl.BlockSpec((1, D), lambda i: (0, 0)),
            pl.BlockSpec((D, 128), lambda i: (0, 0)),
            pl.BlockSpec((1, 128), lambda i: (0, 0)),
        ],
        out_specs=[
            pl.BlockSpec((RB, D), lambda i: (i, 0)),
            pl.BlockSpec((RB, 128), lambda i: (i, 0)),
        ],
        out_shape=[
            jax.ShapeDtypeStruct((N, D), jnp.float32),
            jax.ShapeDtypeStruct((N, 128), jnp.float32),
        ],
    )(agg2, invd2, b2[None, :], wp, bprow)

    direct = small[:, 2:2 + CLS]
    pooler = pooler128[:, 0:CLS]
    loss = _ce(pooler, target) + _ce(direct, target)
    return (out, direct, pooler, loss)


# speedup vs baseline: 2.1152x; 1.3041x over previous
import functools

import jax
import jax.numpy as jnp
from jax import lax
from jax.experimental import pallas as pl
from jax.experimental.pallas import tpu as pltpu

N = 10000
E = 64000
D = 768
IN = 3 * D
CLS = 10
BN_EPS = 1e-5
NEG = 0.2

RB = 1000
WCAT = 1792


def _edge_phase(h, asn, adn, src, dst):
    e = asn[src] + adn[dst]
    e = jnp.where(e > 0, e, NEG * e)
    es = asn + adn
    es = jnp.where(es > 0, es, NEG * es)
    m = jnp.maximum(jnp.max(e), jnp.max(es))
    ex = jnp.exp(e - m)
    exs = jnp.exp(es - m)
    den = jax.ops.segment_sum(ex, dst, num_segments=N) + exs
    pb = h.astype(jnp.bfloat16)[src] * ex.astype(jnp.bfloat16)[:, None]
    agg = jax.ops.segment_sum(pb, dst, num_segments=N).astype(jnp.float32)
    agg = agg + exs[:, None] * h
    return agg, (1.0 / den)[:, None]


def _mm_split_kernel(x_ref, w_ref, r_ref, o1_ref, o2_ref, o3_ref):
    p = jnp.dot(x_ref[...], w_ref[...],
                preferred_element_type=jnp.float32) + r_ref[...]
    o1_ref[...] = p[:, :D]
    o2_ref[...] = p[:, D:D + 128]
    o3_ref[...] = p[:, D + 128:]


def _l2_kernel(p_ref, iv_ref, w_ref, c1_ref, c2_ref, oh_ref, oa_ref):
    a = p_ref[...] * iv_ref[...]
    p = jnp.dot(a, w_ref[...], preferred_element_type=jnp.float32)
    oh_ref[...] = p[:, :D] + c1_ref[...]
    oa_ref[...] = p[:, D:] + c2_ref[...]


def _final_kernel(q_ref, iv_ref, b2_ref, wp_ref, bp_ref, out_ref, pool_ref):
    o = q_ref[...] * iv_ref[...] + b2_ref[...]
    out_ref[...] = o
    pool_ref[...] = jnp.dot(o, wp_ref[...],
                            preferred_element_type=jnp.float32) + bp_ref[...]


def _ce(logits, target):
    mask = (target >= 0)
    tc = jnp.where(mask, target, 0)
    lp = jax.nn.log_softmax(logits, axis=-1)
    nll = -jnp.take_along_axis(lp, tc[:, None], axis=1)[:, 0]
    m = mask.astype(jnp.float32)
    return jnp.sum(nll * m) / jnp.maximum(jnp.sum(m), 1.0)


def kernel(x, edge_index, target, gamma, beta, W1, a1s, a1d, b1,
           W2, a2s, a2d, b2, Wp, bp, Wd, bd):
    src, dst = edge_index[0], edge_index[1]

    mean = jnp.mean(x, axis=0)
    var = jnp.var(x, axis=0)
    s = gamma * lax.rsqrt(var + BN_EPS)
    t = beta - mean * s

    W2a, W2b = W2[:D], W2[D:]
    v1s, v1d = W1 @ a1s, W1 @ a1d
    v2s, v2d = W2b @ a2s, W2b @ a2d
    w3s, w3d = W2a @ a2s, W2a @ a2d

    wcat = jnp.zeros((IN, WCAT), jnp.float32)
    wcat = wcat.at[:, 0:D].set(W1)
    wcat = wcat.at[:, D].set(v1s)
    wcat = wcat.at[:, D + 1].set(v1d)
    wcat = wcat.at[:, D + 2:D + 2 + CLS].set(Wd)
    wcat = wcat.at[:, 896:896 + D].set(W2b)
    wcat = wcat.at[:, 896 + D].set(v2s)
    wcat = wcat.at[:, 896 + D + 1].set(v2d)
    wcat = s[:, None] * wcat

    tW1 = t @ W1
    tW2b = t @ W2b
    row = jnp.zeros((1, WCAT), jnp.float32)
    row = row.at[0, 0:D].set(tW1)
    row = row.at[0, D].set(tW1 @ a1s)
    row = row.at[0, D + 1].set(tW1 @ a1d)
    row = row.at[0, D + 2:D + 2 + CLS].set(t @ Wd + bd)
    row = row.at[0, 896:896 + D].set(tW2b + b1 @ W2a)
    row = row.at[0, 896 + D].set(tW2b @ a2s + b1 @ w3s)
    row = row.at[0, 896 + D + 1].set(tW2b @ a2d + b1 @ w3d)

    h1, small, mid = pl.pallas_call(
        _mm_split_kernel,
        grid=(N // RB,),
        in_specs=[
            pl.BlockSpec((RB, IN), lambda i: (i, 0)),
            pl.BlockSpec((IN, WCAT), lambda i: (0, 0)),
            pl.BlockSpec((1, WCAT), lambda i: (0, 0)),
        ],
        out_specs=[
            pl.BlockSpec((RB, D), lambda i: (i, 0)),
            pl.BlockSpec((RB, 128), lambda i: (i, 0)),
            pl.BlockSpec((RB, 896), lambda i: (i, 0)),
        ],
        out_shape=[
            jax.ShapeDtypeStruct((N, D), jnp.float32),
            jax.ShapeDtypeStruct((N, 128), jnp.float32),
            jax.ShapeDtypeStruct((N, 896), jnp.float32),
        ],
    )(x, wcat, row)

    agg1, invd1 = _edge_phase(h1, small[:, 0], small[:, 1], src, dst)

    w3 = jnp.zeros((D, 896), jnp.float32)
    w3 = w3.at[:, 0:D].set(W2a)
    w3 = w3.at[:, D].set(w3s)
    w3 = w3.at[:, D + 1].set(w3d)

    h2, al2 = pl.pallas_call(
        _l2_kernel,
        grid=(N // RB,),
        in_specs=[
            pl.BlockSpec((RB, D), lambda i: (i, 0)),
            pl.BlockSpec((RB, 1), lambda i: (i, 0)),
            pl.BlockSpec((D, 896), lambda i: (0, 0)),
            pl.BlockSpec((RB, D), lambda i: (i, 0)),
            pl.BlockSpec((RB, 128), lambda i: (i, 0)),
        ],
        out_specs=[
            pl.BlockSpec((RB, D), lambda i: (i, 0)),
            pl.BlockSpec((RB, 128), lambda i: (i, 0)),
        ],
        out_shape=[
            jax.ShapeDtypeStruct((N, D), jnp.float32),
            jax.ShapeDtypeStruct((N, 128), jnp.float32),
        ],
    )(agg1, invd1, w3, mid[:, :D], mid[:, D:])

    agg2, invd2 = _edge_phase(h2, al2[:, 0], al2[:, 1], src, dst)

    wp = jnp.zeros((D, 128), jnp.float32).at[:, 0:CLS].set(Wp)
    bprow = jnp.zeros((1, 128), jnp.float32).at[0, 0:CLS].set(bp)

    out, pooler128 = pl.pallas_call(
        _final_kernel,
        grid=(N // RB,),
        in_specs=[
            pl.BlockSpec((RB, D), lambda i: (i, 0)),
            pl.BlockSpec((RB, 1), lambda i: (i, 0)),
            pl.BlockSpec((1, D), lambda i: (0, 0)),
            pl.BlockSpec((D, 128), lambda i: (0, 0)),
            pl.BlockSpec((1, 128), lambda i: (0, 0)),
        ],
        out_specs=[
            pl.BlockSpec((RB, D), lambda i: (i, 0)),
            pl.BlockSpec((RB, 128), lambda i: (i, 0)),
        ],
        out_shape=[
            jax.ShapeDtypeStruct((N, D), jnp.float32),
            jax.ShapeDtypeStruct((N, 128), jnp.float32),
        ],
    )(agg2, invd2, b2[None, :], wp, bprow)

    direct = small[:, 2:2 + CLS]
    pooler = pooler128[:, 0:CLS]
    loss = _ce(pooler, target) + _ce(direct, target)
    return (out, direct, pooler, loss)
